# baseline (speedup 1.0000x reference)
import functools

import jax
import jax.numpy as jnp
from jax import lax
from jax.experimental import pallas as pl
from jax.experimental.pallas import tpu as pltpu
from jax.experimental.pallas import tpu_sc as plsc

SEQ = 12
IMG = 32
EMB_D = IMG * IMG
X_CH = 3
X_COLS = X_CH * SEQ * EMB_D
O_COLS = (X_CH + 1) * SEQ * EMB_D


def _sc_gather(table, ids):
    n_rows, d = table.shape
    b = ids.shape[0]
    info = plsc.get_sparse_core_info()
    nw = info.num_cores * info.num_subcores
    b_per_w = b // nw

    mesh = plsc.VectorSubcoreMesh(core_axis_name="c", subcore_axis_name="s")

    @functools.partial(
        pl.kernel,
        mesh=mesh,
        out_type=jax.ShapeDtypeStruct((b, d), jnp.float32),
        scratch_types=[
            pltpu.VMEM((b_per_w,), jnp.int32),
            pltpu.VMEM((b_per_w, d), jnp.float32),
            pltpu.SemaphoreType.DMA,
        ],
    )
    def gather_kernel(table_hbm, idx_hbm, out_hbm, idx_v, rows_v, sem):
        wid = lax.axis_index("s") * info.num_cores + lax.axis_index("c")
        base = wid * b_per_w
        pltpu.sync_copy(idx_hbm.at[pl.ds(base, b_per_w)], idx_v)
        pltpu.async_copy(table_hbm.at[idx_v], rows_v, sem).wait()
        pltpu.sync_copy(rows_v, out_hbm.at[pl.ds(base, b_per_w)])

    return gather_kernel(table, ids)


def _copy_x(xt):
    b = xt.shape[1]
    rows = 1024
    n_xblk = X_COLS // rows

    def body(x_ref, o_ref):
        o_ref[...] = x_ref[...]

    return pl.pallas_call(
        body,
        grid=(n_xblk,),
        in_specs=[pl.BlockSpec((rows, b), lambda i: (i, 0))],
        out_specs=pl.BlockSpec((rows, b), lambda i: (i, 0)),
        out_shape=jax.ShapeDtypeStruct((O_COLS, b), jnp.float32),
    )(xt)


def _fill_emb(buf, emb):
    b = emb.shape[0]
    reps = 2
    rows = reps * EMB_D
    n_xblk = X_COLS // rows

    def body(buf_ref, e_ref, o_ref, et_ref):
        @pl.when(pl.program_id(0) == 0)
        def _():
            et_ref[...] = e_ref[...].T

        e = et_ref[...]
        for k in range(reps):
            o_ref[k * EMB_D:(k + 1) * EMB_D, :] = e

    return pl.pallas_call(
        body,
        grid=(SEQ // reps,),
        in_specs=[
            pl.BlockSpec(memory_space=pl.ANY),
            pl.BlockSpec((b, EMB_D), lambda j: (0, 0)),
        ],
        out_specs=pl.BlockSpec((rows, b), lambda j: (n_xblk + j, 0)),
        out_shape=jax.ShapeDtypeStruct((O_COLS, b), jnp.float32),
        input_output_aliases={0: 0},
        scratch_shapes=[pltpu.VMEM((EMB_D, b), jnp.float32)],
    )(buf, emb)


def kernel(x, id, table):
    b = x.shape[0]
    emb = _sc_gather(table, id)
    xt = x.reshape(b, X_COLS).T
    buf = _copy_x(xt)
    outt = _fill_emb(buf, emb)
    return outt.T.reshape(b, X_CH + 1, SEQ, IMG, IMG)


# speedup vs baseline: 1.0191x; 1.0191x over previous
import functools

import jax
import jax.numpy as jnp
from jax import lax
from jax.experimental import pallas as pl
from jax.experimental.pallas import tpu as pltpu
from jax.experimental.pallas import tpu_sc as plsc

SEQ = 12
IMG = 32
EMB_D = IMG * IMG
X_CH = 3
X_COLS = X_CH * SEQ * EMB_D
O_COLS = (X_CH + 1) * SEQ * EMB_D


def _sc_gather(table, ids):
    n_rows, d = table.shape
    b = ids.shape[0]
    info = plsc.get_sparse_core_info()
    nw = info.num_cores * info.num_subcores
    b_per_w = b // nw

    mesh = plsc.VectorSubcoreMesh(core_axis_name="c", subcore_axis_name="s")

    @functools.partial(
        pl.kernel,
        mesh=mesh,
        out_type=jax.ShapeDtypeStruct((b, d), jnp.float32),
        scratch_types=[
            pltpu.VMEM((b_per_w,), jnp.int32),
            pltpu.VMEM((b_per_w, d), jnp.float32),
            pltpu.SemaphoreType.DMA,
        ],
    )
    def gather_kernel(table_hbm, idx_hbm, out_hbm, idx_v, rows_v, sem):
        wid = lax.axis_index("s") * info.num_cores + lax.axis_index("c")
        base = wid * b_per_w
        pltpu.sync_copy(idx_hbm.at[pl.ds(base, b_per_w)], idx_v)
        pltpu.async_copy(table_hbm.at[idx_v], rows_v, sem).wait()
        pltpu.sync_copy(rows_v, out_hbm.at[pl.ds(base, b_per_w)])

    return gather_kernel(table, ids)


def _copy_x(xt):
    b = xt.shape[1]
    rows = 3072
    n_xblk = X_COLS // rows

    def body(x_ref, o_ref):
        o_ref[...] = x_ref[...]

    return pl.pallas_call(
        body,
        grid=(n_xblk,),
        in_specs=[pl.BlockSpec((rows, b), lambda i: (i, 0))],
        out_specs=pl.BlockSpec((rows, b), lambda i: (i, 0)),
        out_shape=jax.ShapeDtypeStruct((O_COLS, b), jnp.float32),
    )(xt)


def _fill_emb(buf, emb):
    b = emb.shape[0]
    reps = 2
    rows = reps * EMB_D
    n_xblk = X_COLS // rows

    def body(buf_ref, e_ref, o_ref, et_ref):
        @pl.when(pl.program_id(0) == 0)
        def _():
            et_ref[...] = e_ref[...].T

        e = et_ref[...]
        for k in range(reps):
            o_ref[k * EMB_D:(k + 1) * EMB_D, :] = e

    return pl.pallas_call(
        body,
        grid=(SEQ // reps,),
        in_specs=[
            pl.BlockSpec(memory_space=pl.ANY),
            pl.BlockSpec((b, EMB_D), lambda j: (0, 0)),
        ],
        out_specs=pl.BlockSpec((rows, b), lambda j: (n_xblk + j, 0)),
        out_shape=jax.ShapeDtypeStruct((O_COLS, b), jnp.float32),
        input_output_aliases={0: 0},
        scratch_shapes=[pltpu.VMEM((EMB_D, b), jnp.float32)],
    )(buf, emb)


def kernel(x, id, table):
    b = x.shape[0]
    emb = _sc_gather(table, id)
    xt = x.reshape(b, X_COLS).T
    buf = _copy_x(xt)
    outt = _fill_emb(buf, emb)
    return outt.T.reshape(b, X_CH + 1, SEQ, IMG, IMG)
